# baseline (speedup 1.0000x reference)
import functools

import jax
import jax.numpy as jnp
from jax import lax
from jax.experimental import pallas as pl
from jax.experimental.pallas import tpu as pltpu
from jax.experimental.pallas import tpu_sc as plsc

D = 128
K = 128
NC = 2
NS = 16
NW = NC * NS
NB = 4


def _ceil_to(x, m):
    return ((x + m - 1) // m) * m


@functools.lru_cache(maxsize=None)
def _make_feat2(e1, acc1, e2, acc2, c0_frac_num=1, c0_frac_den=2):
    def _split(e):
        tot = e // (NS * K)
        q0 = _ceil_to((tot * c0_frac_num) // c0_frac_den, NB)
        return q0, tot - q0
    qs = (_split(e1), _split(e2))
    zs = (acc1 // NS, acc2 // NS)
    accmax = max(acc1, acc2)
    mesh = plsc.VectorSubcoreMesh(core_axis_name="c", subcore_axis_name="s")
    out_type = [
        jax.ShapeDtypeStruct((NC, acc1, D), jnp.float32),
        jax.ShapeDtypeStruct((NC, acc2, D), jnp.float32),
        jax.ShapeDtypeStruct((NC, NS, acc1), jnp.float32),
        jax.ShapeDtypeStruct((NC, NS, acc2), jnp.float32),
    ]
    scratch = [
        pltpu.VMEM_SHARED((accmax, D), jnp.float32),
        pltpu.VMEM((NB, K), jnp.int32),
        pltpu.VMEM((NB, K), jnp.int32),
        pltpu.VMEM((K, D), jnp.float32),
        pltpu.VMEM((K, D), jnp.float32),
        pltpu.VMEM((accmax,), jnp.float32),
        pltpu.SemaphoreType.DMA,
        pltpu.SemaphoreType.DMA,
        pltpu.SemaphoreType.DMA,
        pltpu.SemaphoreType.DMA,
    ]

    @functools.partial(
        pl.kernel, mesh=mesh, out_type=out_type, scratch_types=scratch,
        compiler_params=pltpu.CompilerParams(needs_layout_passes=False))
    def feat2(t1, s1, d1, t2, s2, d2, zf,
              sum1, sum2, deg1, deg2,
              acc, sblk, dblk, rows0, rows1, hist, sg0, sg1, ss0, ss1):
        c = lax.axis_index("c")
        s = lax.axis_index("s")
        wid = s * NC + c
        ones16 = jnp.ones((16,), jnp.float32)
        zero16 = jnp.zeros((16,), jnp.float32)

        for (table, srcr, dstr, (q0, q1), z, acc_n, sumo, dego) in (
                (t1, s1, d1, qs[0], zs[0], acc1, sum1, deg1),
                (t2, s2, d2, qs[1], zs[1], acc2, sum2, deg2)):
            nzc = z // K
            pltpu.sync_copy(zf, rows0)

            def zbody(i, carry):
                pltpu.sync_copy(rows0, acc.at[pl.ds(s * z + i * K, K)])
                return carry

            lax.fori_loop(0, nzc, zbody, 0)

            def zhist(i, carry):
                hist[pl.ds(i * 16, 16)] = zero16
                return carry

            lax.fori_loop(0, acc_n // 16, zhist, 0)
            plsc.subcore_barrier()
            crow0 = jnp.where(c == 0, s * q0, NS * q0 + s * q1)
            nblk = jnp.where(c == 0, q0 // NB, q1 // NB)

            def body(t, carry):
                crow = crow0 + t * NB
                i0 = pltpu.async_copy(srcr.at[pl.ds(crow, NB)], sblk, sg0)
                i1 = pltpu.async_copy(dstr.at[pl.ds(crow, NB)], dblk, sg1)
                i0.wait()
                i1.wait()
                cprev = None
                for h in range(NB // 2):
                    if cprev is not None:
                        cprev[0].wait()
                    g0 = pltpu.async_copy(table.at[sblk.at[2 * h]],
                                          rows0, sg0)
                    if cprev is not None:
                        cprev[1].wait()
                    g1 = pltpu.async_copy(table.at[sblk.at[2 * h + 1]],
                                          rows1, sg1)
                    for u in (2 * h, 2 * h + 1):
                        for j in range(K // 16):
                            idx16 = dblk[u, pl.ds(j * 16, 16)]
                            plsc.addupdate_scatter(hist, [idx16], ones16)
                    g0.wait()
                    c0 = pltpu.async_copy(rows0, acc.at[dblk.at[2 * h]],
                                          ss0, add=True)
                    g1.wait()
                    c1 = pltpu.async_copy(rows1, acc.at[dblk.at[2 * h + 1]],
                                          ss1, add=True)
                    cprev = (c0, c1)
                cprev[0].wait()
                cprev[1].wait()
                return carry

            lax.fori_loop(0, nblk, body, 0)
            plsc.subcore_barrier()
            pltpu.sync_copy(hist.at[pl.ds(0, acc_n)], dego.at[c, s])

            def obody(i, carry):
                r0 = s * z + i * K
                pltpu.sync_copy(acc.at[pl.ds(r0, K)], rows0)
                pltpu.sync_copy(rows0, sumo.at[c, pl.ds(r0, K)])
                return carry

            lax.fori_loop(0, nzc, obody, 0)
            plsc.subcore_barrier()

    return feat2


def _pad_edges(ei, n_dst):
    src, dst = ei[0], ei[1]
    e = src.shape[0]
    epad = _ceil_to(e, NW * K * NB)
    pad = epad - e
    if pad:
        src = jnp.concatenate([src, jnp.zeros((pad,), jnp.int32)])
        dst = jnp.concatenate([dst, jnp.full((pad,), n_dst, jnp.int32)])
    return src.reshape(epad // K, K), dst.reshape(epad // K, K), epad


def _mean_linear(sums, degs, W, b, n, leaky):
    blk = 1000
    nb = n // blk

    def body(s_ref, d_ref, w_ref, b_ref, o_ref):
        ss = s_ref[...]
        dd = d_ref[...]
        sm = ss[0] + ss[1]
        d = jnp.sum(dd, axis=1)[:, None]
        mean = sm / jnp.maximum(d, 1.0)
        h = jnp.dot(mean, w_ref[...], preferred_element_type=jnp.float32)
        h = h + jnp.where(d > 0, b_ref[...], 0.0)
        if leaky:
            h = jnp.where(h >= 0, h, 0.01 * h)
        o_ref[...] = h

    return pl.pallas_call(
        body,
        grid=(nb,),
        in_specs=[
            pl.BlockSpec((NC, blk, D), lambda i: (0, i, 0)),
            pl.BlockSpec((blk, NW), lambda i: (i, 0)),
            pl.BlockSpec((D, D), lambda i: (0, 0)),
            pl.BlockSpec((1, D), lambda i: (0, 0)),
        ],
        out_specs=pl.BlockSpec((blk, D), lambda i: (i, 0)),
        out_shape=jax.ShapeDtypeStruct((n, D), jnp.float32),
    )(sums, degs, W, b.reshape(1, D))


def _final_combine(sw, dw, Ww, bw, st, dt, Wt, bt, n):
    blk = 1000
    nb = n // blk

    def body(sw_ref, dw_ref, ww_ref, bw_ref, st_ref, dt_ref, wt_ref, bt_ref,
             o_ref):
        out = None
        for s_ref, d_ref, w_ref, b_ref in (
                (sw_ref, dw_ref, ww_ref, bw_ref),
                (st_ref, dt_ref, wt_ref, bt_ref)):
            ss = s_ref[...]
            dd = d_ref[...]
            sm = ss[0] + ss[1]
            d = jnp.sum(dd, axis=1)[:, None]
            mean = sm / jnp.maximum(d, 1.0)
            h = jnp.dot(mean, w_ref[...], preferred_element_type=jnp.float32)
            h = h + jnp.where(d > 0, b_ref[...], 0.0)
            out = h if out is None else out + h
        o_ref[...] = out

    mat = pl.BlockSpec((NC, blk, D), lambda i: (0, i, 0))
    deg = pl.BlockSpec((blk, NW), lambda i: (i, 0))
    wsp = pl.BlockSpec((D, D), lambda i: (0, 0))
    bsp = pl.BlockSpec((1, D), lambda i: (0, 0))
    return pl.pallas_call(
        body,
        grid=(nb,),
        in_specs=[mat, deg, wsp, bsp, mat, deg, wsp, bsp],
        out_specs=pl.BlockSpec((blk, D), lambda i: (i, 0)),
        out_shape=jax.ShapeDtypeStruct((n, D), jnp.float32),
    )(sw, dw, Ww, bw.reshape(1, D), st, dt, Wt, bt.reshape(1, D))


def kernel(embeds, params, edges):
    paper = embeds["paper"]
    n_author, n_paper, n_field = 10000, 10000, 5000

    W_wb, b_wb = params["layer0"]["paper,written_by,author"]
    W_ht, b_ht = params["layer0"]["paper,has_topic,field"]
    W_w, b_w = params["layer1"]["author,writes,paper"]
    W_t, b_t = params["layer1"]["field,topic_of,paper"]

    s_wb, d_wb, e_wb = _pad_edges(edges["paper,written_by,author"], n_author)
    s_ht, d_ht, e_ht = _pad_edges(edges["paper,has_topic,field"], n_field)
    s_w, d_w, e_w = _pad_edges(edges["author,writes,paper"], n_paper)
    s_t, d_t, e_t = _pad_edges(edges["field,topic_of,paper"], n_paper)

    acc_a = _ceil_to(n_author + 8, NS * K)
    acc_f = _ceil_to(n_field + 8, NS * K)
    acc_p = _ceil_to(n_paper + 8, NS * K)
    zf = jnp.zeros((K, D), jnp.float32)

    sum_a, sum_f, deg_a, deg_f = _make_feat2(e_wb, acc_a, e_ht, acc_f, 3, 10)(
        paper, s_wb, d_wb, paper, s_ht, d_ht, zf)
    h_a = _mean_linear(sum_a[:, :n_author],
                       deg_a.reshape(NW, -1).T[:n_author],
                       W_wb, b_wb, n_author, leaky=True)
    h_f = _mean_linear(sum_f[:, :n_field],
                       deg_f.reshape(NW, -1).T[:n_field],
                       W_ht, b_ht, n_field, leaky=True)

    sum_w, sum_t, deg_w, deg_t = _make_feat2(e_w, acc_p, e_t, acc_p, 3, 10)(
        h_a, s_w, d_w, h_f, s_t, d_t, zf)
    return _final_combine(sum_w[:, :n_paper],
                          deg_w.reshape(NW, -1).T[:n_paper], W_w, b_w,
                          sum_t[:, :n_paper],
                          deg_t.reshape(NW, -1).T[:n_paper], W_t, b_t,
                          n_paper)


# speedup vs baseline: 1.1153x; 1.1153x over previous
import functools

import jax
import jax.numpy as jnp
from jax import lax
from jax.experimental import pallas as pl
from jax.experimental.pallas import tpu as pltpu
from jax.experimental.pallas import tpu_sc as plsc

D = 128
K = 128
NC = 2
NS = 16
NW = NC * NS
NB = 4


def _ceil_to(x, m):
    return ((x + m - 1) // m) * m


@functools.lru_cache(maxsize=None)
def _make_feat2(e1, acc1, e2, acc2, c0_frac_num=1, c0_frac_den=2):
    def _split(e):
        tot = e // (NS * K)
        q0 = _ceil_to((tot * c0_frac_num) // c0_frac_den, NB)
        return q0, tot - q0
    qs = (_split(e1), _split(e2))
    zs = (acc1 // NS, acc2 // NS)
    accmax = max(acc1, acc2)
    mesh = plsc.VectorSubcoreMesh(core_axis_name="c", subcore_axis_name="s")
    out_type = [
        jax.ShapeDtypeStruct((NC, acc1, D), jnp.float32),
        jax.ShapeDtypeStruct((NC, acc2, D), jnp.float32),
        jax.ShapeDtypeStruct((NC, NS, acc1), jnp.float32),
        jax.ShapeDtypeStruct((NC, NS, acc2), jnp.float32),
    ]
    scratch = [
        pltpu.VMEM_SHARED((accmax, D), jnp.float32),
        pltpu.VMEM((NB, K), jnp.int32),
        pltpu.VMEM((NB, K), jnp.int32),
        pltpu.VMEM((K, D), jnp.float32),
        pltpu.VMEM((K, D), jnp.float32),
        pltpu.VMEM((accmax,), jnp.float32),
        pltpu.SemaphoreType.DMA,
        pltpu.SemaphoreType.DMA,
        pltpu.SemaphoreType.DMA,
        pltpu.SemaphoreType.DMA,
    ]

    @functools.partial(
        pl.kernel, mesh=mesh, out_type=out_type, scratch_types=scratch,
        compiler_params=pltpu.CompilerParams(needs_layout_passes=False))
    def feat2(t1, s1, d1, t2, s2, d2, zf,
              sum1, sum2, deg1, deg2,
              acc, sblk, dblk, rows0, rows1, hist, sg0, sg1, ss0, ss1):
        c = lax.axis_index("c")
        s = lax.axis_index("s")
        wid = s * NC + c
        ones16 = jnp.ones((16,), jnp.float32)
        zero16 = jnp.zeros((16,), jnp.float32)

        for (table, srcr, dstr, (q0, q1), z, acc_n, sumo, dego) in (
                (t1, s1, d1, qs[0], zs[0], acc1, sum1, deg1),
                (t2, s2, d2, qs[1], zs[1], acc2, sum2, deg2)):
            nzc = z // K
            pltpu.sync_copy(zf, rows0)

            def zbody(i, carry):
                pltpu.sync_copy(rows0, acc.at[pl.ds(s * z + i * K, K)])
                return carry

            lax.fori_loop(0, nzc, zbody, 0)

            def zhist(i, carry):
                hist[pl.ds(i * 16, 16)] = zero16
                return carry

            lax.fori_loop(0, acc_n // 16, zhist, 0)
            plsc.subcore_barrier()
            crow0 = jnp.where(c == 0, s * q0, NS * q0 + s * q1)
            nblk = jnp.where(c == 0, q0 // NB, q1 // NB)

            def body(t, carry):
                crow = crow0 + t * NB
                i0 = pltpu.async_copy(srcr.at[pl.ds(crow, NB)], sblk, sg0)
                i1 = pltpu.async_copy(dstr.at[pl.ds(crow, NB)], dblk, sg1)
                i0.wait()
                i1.wait()
                cprev = None
                for h in range(NB // 2):
                    if cprev is not None:
                        cprev[0].wait()
                    g0 = pltpu.async_copy(table.at[sblk.at[2 * h]],
                                          rows0, sg0)
                    if cprev is not None:
                        cprev[1].wait()
                    g1 = pltpu.async_copy(table.at[sblk.at[2 * h + 1]],
                                          rows1, sg1)
                    for u in (2 * h, 2 * h + 1):
                        for j in range(K // 16):
                            idx16 = dblk[u, pl.ds(j * 16, 16)]
                            plsc.addupdate_scatter(hist, [idx16], ones16)
                    g0.wait()
                    c0 = pltpu.async_copy(rows0, acc.at[dblk.at[2 * h]],
                                          ss0, add=True)
                    g1.wait()
                    c1 = pltpu.async_copy(rows1, acc.at[dblk.at[2 * h + 1]],
                                          ss1, add=True)
                    cprev = (c0, c1)
                cprev[0].wait()
                cprev[1].wait()
                return carry

            lax.fori_loop(0, nblk, body, 0)
            plsc.subcore_barrier()
            pltpu.sync_copy(hist.at[pl.ds(0, acc_n)], dego.at[c, s])

            def obody(i, carry):
                r0 = s * z + i * K
                pltpu.sync_copy(acc.at[pl.ds(r0, K)], rows0)
                pltpu.sync_copy(rows0, sumo.at[c, pl.ds(r0, K)])
                return carry

            lax.fori_loop(0, nzc, obody, 0)
            plsc.subcore_barrier()

    return feat2


def _pad_edges(ei, n_dst):
    src, dst = ei[0], ei[1]
    e = src.shape[0]
    epad = _ceil_to(e, NW * K * NB)
    pad = epad - e
    if pad:
        src = jnp.concatenate([src, jnp.zeros((pad,), jnp.int32)])
        dst = jnp.concatenate([dst, jnp.full((pad,), n_dst, jnp.int32)])
    return src.reshape(epad // K, K), dst.reshape(epad // K, K), epad


def _mean_linear(sums, degs, W, b, n, leaky):
    blk = 1000
    nb = n // blk

    def body(s_ref, d_ref, w_ref, b_ref, o_ref):
        ss = s_ref[...]
        dd = d_ref[...]
        sm = ss[0] + ss[1]
        d = jnp.sum(dd, axis=1)[:, None]
        mean = sm / jnp.maximum(d, 1.0)
        h = jnp.dot(mean, w_ref[...], preferred_element_type=jnp.float32)
        h = h + jnp.where(d > 0, b_ref[...], 0.0)
        if leaky:
            h = jnp.where(h >= 0, h, 0.01 * h)
        o_ref[...] = h

    return pl.pallas_call(
        body,
        grid=(nb,),
        in_specs=[
            pl.BlockSpec((NC, blk, D), lambda i: (0, i, 0)),
            pl.BlockSpec((blk, NW), lambda i: (i, 0)),
            pl.BlockSpec((D, D), lambda i: (0, 0)),
            pl.BlockSpec((1, D), lambda i: (0, 0)),
        ],
        out_specs=pl.BlockSpec((blk, D), lambda i: (i, 0)),
        out_shape=jax.ShapeDtypeStruct((n, D), jnp.float32),
    )(sums, degs, W, b.reshape(1, D))


def _final_combine(sw, dw, Ww, bw, st, dt, Wt, bt, n):
    blk = 1000
    nb = n // blk

    def body(sw_ref, dw_ref, ww_ref, bw_ref, st_ref, dt_ref, wt_ref, bt_ref,
             o_ref):
        out = None
        for s_ref, d_ref, w_ref, b_ref in (
                (sw_ref, dw_ref, ww_ref, bw_ref),
                (st_ref, dt_ref, wt_ref, bt_ref)):
            ss = s_ref[...]
            dd = d_ref[...]
            sm = ss[0] + ss[1]
            d = jnp.sum(dd, axis=1)[:, None]
            mean = sm / jnp.maximum(d, 1.0)
            h = jnp.dot(mean, w_ref[...], preferred_element_type=jnp.float32)
            h = h + jnp.where(d > 0, b_ref[...], 0.0)
            out = h if out is None else out + h
        o_ref[...] = out

    mat = pl.BlockSpec((NC, blk, D), lambda i: (0, i, 0))
    deg = pl.BlockSpec((blk, NW), lambda i: (i, 0))
    wsp = pl.BlockSpec((D, D), lambda i: (0, 0))
    bsp = pl.BlockSpec((1, D), lambda i: (0, 0))
    return pl.pallas_call(
        body,
        grid=(nb,),
        in_specs=[mat, deg, wsp, bsp, mat, deg, wsp, bsp],
        out_specs=pl.BlockSpec((blk, D), lambda i: (i, 0)),
        out_shape=jax.ShapeDtypeStruct((n, D), jnp.float32),
    )(sw, dw, Ww, bw.reshape(1, D), st, dt, Wt, bt.reshape(1, D))


def kernel(embeds, params, edges):
    paper = embeds["paper"]
    n_author, n_paper, n_field = 10000, 10000, 5000

    W_wb, b_wb = params["layer0"]["paper,written_by,author"]
    W_ht, b_ht = params["layer0"]["paper,has_topic,field"]
    W_w, b_w = params["layer1"]["author,writes,paper"]
    W_t, b_t = params["layer1"]["field,topic_of,paper"]

    s_wb, d_wb, e_wb = _pad_edges(edges["paper,written_by,author"], n_author)
    s_ht, d_ht, e_ht = _pad_edges(edges["paper,has_topic,field"], n_field)
    s_w, d_w, e_w = _pad_edges(edges["author,writes,paper"], n_paper)
    s_t, d_t, e_t = _pad_edges(edges["field,topic_of,paper"], n_paper)

    acc_a = _ceil_to(n_author + 8, NS * K)
    acc_f = _ceil_to(n_field + 8, NS * K)
    acc_p = _ceil_to(n_paper + 8, NS * K)
    zf = jnp.zeros((K, D), jnp.float32)

    sum_a, sum_f, deg_a, deg_f = _make_feat2(e_wb, acc_a, e_ht, acc_f, 3, 4)(
        paper, s_wb, d_wb, paper, s_ht, d_ht, zf)
    h_a = _mean_linear(sum_a[:, :n_author],
                       deg_a.reshape(NW, -1).T[:n_author],
                       W_wb, b_wb, n_author, leaky=True)
    h_f = _mean_linear(sum_f[:, :n_field],
                       deg_f.reshape(NW, -1).T[:n_field],
                       W_ht, b_ht, n_field, leaky=True)

    sum_w, sum_t, deg_w, deg_t = _make_feat2(e_w, acc_p, e_t, acc_p, 3, 4)(
        h_a, s_w, d_w, h_f, s_t, d_t, zf)
    return _final_combine(sum_w[:, :n_paper],
                          deg_w.reshape(NW, -1).T[:n_paper], W_w, b_w,
                          sum_t[:, :n_paper],
                          deg_t.reshape(NW, -1).T[:n_paper], W_t, b_t,
                          n_paper)


# speedup vs baseline: 1.1529x; 1.0337x over previous
import functools

import jax
import jax.numpy as jnp
from jax import lax
from jax.experimental import pallas as pl
from jax.experimental.pallas import tpu as pltpu
from jax.experimental.pallas import tpu_sc as plsc

D = 128
K = 128
NC = 2
NS = 16
NW = NC * NS
NB = 4


def _ceil_to(x, m):
    return ((x + m - 1) // m) * m


@functools.lru_cache(maxsize=None)
def _make_feat2(e1, acc1, e2, acc2, c0_frac_num=1, c0_frac_den=2):
    def _split(e):
        tot = e // (NS * K)
        q0 = _ceil_to((tot * c0_frac_num) // c0_frac_den, NB)
        return q0, tot - q0
    qs = (_split(e1), _split(e2))
    zs = (acc1 // NS, acc2 // NS)
    accmax = max(acc1, acc2)
    mesh = plsc.VectorSubcoreMesh(core_axis_name="c", subcore_axis_name="s")
    out_type = [
        jax.ShapeDtypeStruct((NC, acc1, D), jnp.float32),
        jax.ShapeDtypeStruct((NC, acc2, D), jnp.float32),
        jax.ShapeDtypeStruct((NC, NS, acc1), jnp.float32),
        jax.ShapeDtypeStruct((NC, NS, acc2), jnp.float32),
    ]
    scratch = [
        pltpu.VMEM_SHARED((accmax, D), jnp.float32),
        pltpu.VMEM((NB, K), jnp.int32),
        pltpu.VMEM((NB, K), jnp.int32),
        pltpu.VMEM((K, D), jnp.float32),
        pltpu.VMEM((K, D), jnp.float32),
        pltpu.VMEM((accmax,), jnp.float32),
        pltpu.SemaphoreType.DMA,
        pltpu.SemaphoreType.DMA,
        pltpu.SemaphoreType.DMA,
        pltpu.SemaphoreType.DMA,
    ]

    @functools.partial(
        pl.kernel, mesh=mesh, out_type=out_type, scratch_types=scratch,
        compiler_params=pltpu.CompilerParams(needs_layout_passes=False))
    def feat2(t1, s1, d1, t2, s2, d2, zf,
              sum1, sum2, deg1, deg2,
              acc, sblk, dblk, rows0, rows1, hist, sg0, sg1, ss0, ss1):
        c = lax.axis_index("c")
        s = lax.axis_index("s")
        wid = s * NC + c
        ones16 = jnp.ones((16,), jnp.float32)
        zero16 = jnp.zeros((16,), jnp.float32)

        for (table, srcr, dstr, (q0, q1), z, acc_n, sumo, dego) in (
                (t1, s1, d1, qs[0], zs[0], acc1, sum1, deg1),
                (t2, s2, d2, qs[1], zs[1], acc2, sum2, deg2)):
            nzc = z // K
            pltpu.sync_copy(zf, rows0)

            def zbody(i, carry):
                pltpu.sync_copy(rows0, acc.at[pl.ds(s * z + i * K, K)])
                return carry

            lax.fori_loop(0, nzc, zbody, 0)

            def zhist(i, carry):
                hist[pl.ds(i * 16, 16)] = zero16
                return carry

            lax.fori_loop(0, acc_n // 16, zhist, 0)
            plsc.subcore_barrier()
            crow0 = jnp.where(c == 0, s * q0, NS * q0 + s * q1)
            nblk = jnp.where(c == 0, q0 // NB, q1 // NB)

            def body(t, carry):
                crow = crow0 + t * NB
                i0 = pltpu.async_copy(srcr.at[pl.ds(crow, NB)], sblk, sg0)
                i1 = pltpu.async_copy(dstr.at[pl.ds(crow, NB)], dblk, sg1)
                i0.wait()
                i1.wait()
                cprev = None
                for h in range(NB // 2):
                    if cprev is not None:
                        cprev[0].wait()
                    g0 = pltpu.async_copy(table.at[sblk.at[2 * h]],
                                          rows0, sg0)
                    if cprev is not None:
                        cprev[1].wait()
                    g1 = pltpu.async_copy(table.at[sblk.at[2 * h + 1]],
                                          rows1, sg1)
                    for u in (2 * h, 2 * h + 1):
                        for j in range(K // 16):
                            idx16 = dblk[u, pl.ds(j * 16, 16)]
                            plsc.addupdate_scatter(hist, [idx16], ones16)
                    g0.wait()
                    c0 = pltpu.async_copy(rows0, acc.at[dblk.at[2 * h]],
                                          ss0, add=True)
                    g1.wait()
                    c1 = pltpu.async_copy(rows1, acc.at[dblk.at[2 * h + 1]],
                                          ss1, add=True)
                    cprev = (c0, c1)
                cprev[0].wait()
                cprev[1].wait()
                return carry

            lax.fori_loop(0, nblk, body, 0)
            plsc.subcore_barrier()
            pltpu.sync_copy(hist.at[pl.ds(0, acc_n)], dego.at[c, s])

            def obody(i, carry):
                r0 = s * z + i * K
                pltpu.sync_copy(acc.at[pl.ds(r0, K)], rows0)
                pltpu.sync_copy(rows0, sumo.at[c, pl.ds(r0, K)])
                return carry

            lax.fori_loop(0, nzc, obody, 0)
            plsc.subcore_barrier()

    return feat2


def _pad_edges(ei, n_dst):
    src, dst = ei[0], ei[1]
    e = src.shape[0]
    epad = _ceil_to(e, NW * K * NB)
    pad = epad - e
    if pad:
        src = jnp.concatenate([src, jnp.zeros((pad,), jnp.int32)])
        dst = jnp.concatenate([dst, jnp.full((pad,), n_dst, jnp.int32)])
    return src.reshape(epad // K, K), dst.reshape(epad // K, K), epad


def _mean_linear(sums, degs, W, b, n, leaky):
    blk = 1000
    nb = n // blk

    def body(s_ref, d_ref, w_ref, b_ref, o_ref):
        ss = s_ref[...]
        dd = d_ref[...]
        sm = ss[0] + ss[1]
        d = jnp.sum(dd, axis=1)[:, None]
        mean = sm / jnp.maximum(d, 1.0)
        h = jnp.dot(mean, w_ref[...], preferred_element_type=jnp.float32)
        h = h + jnp.where(d > 0, b_ref[...], 0.0)
        if leaky:
            h = jnp.where(h >= 0, h, 0.01 * h)
        o_ref[...] = h

    return pl.pallas_call(
        body,
        grid=(nb,),
        in_specs=[
            pl.BlockSpec((NC, blk, D), lambda i: (0, i, 0)),
            pl.BlockSpec((blk, NW), lambda i: (i, 0)),
            pl.BlockSpec((D, D), lambda i: (0, 0)),
            pl.BlockSpec((1, D), lambda i: (0, 0)),
        ],
        out_specs=pl.BlockSpec((blk, D), lambda i: (i, 0)),
        out_shape=jax.ShapeDtypeStruct((n, D), jnp.float32),
    )(sums, degs, W, b.reshape(1, D))


def _final_combine(sw, dw, Ww, bw, st, dt, Wt, bt, n):
    blk = 1000
    nb = n // blk

    def body(sw_ref, dw_ref, ww_ref, bw_ref, st_ref, dt_ref, wt_ref, bt_ref,
             o_ref):
        out = None
        for s_ref, d_ref, w_ref, b_ref in (
                (sw_ref, dw_ref, ww_ref, bw_ref),
                (st_ref, dt_ref, wt_ref, bt_ref)):
            ss = s_ref[...]
            dd = d_ref[...]
            sm = ss[0] + ss[1]
            d = jnp.sum(dd, axis=1)[:, None]
            mean = sm / jnp.maximum(d, 1.0)
            h = jnp.dot(mean, w_ref[...], preferred_element_type=jnp.float32)
            h = h + jnp.where(d > 0, b_ref[...], 0.0)
            out = h if out is None else out + h
        o_ref[...] = out

    mat = pl.BlockSpec((NC, blk, D), lambda i: (0, i, 0))
    deg = pl.BlockSpec((blk, NW), lambda i: (i, 0))
    wsp = pl.BlockSpec((D, D), lambda i: (0, 0))
    bsp = pl.BlockSpec((1, D), lambda i: (0, 0))
    return pl.pallas_call(
        body,
        grid=(nb,),
        in_specs=[mat, deg, wsp, bsp, mat, deg, wsp, bsp],
        out_specs=pl.BlockSpec((blk, D), lambda i: (i, 0)),
        out_shape=jax.ShapeDtypeStruct((n, D), jnp.float32),
    )(sw, dw, Ww, bw.reshape(1, D), st, dt, Wt, bt.reshape(1, D))


def kernel(embeds, params, edges):
    paper = embeds["paper"]
    n_author, n_paper, n_field = 10000, 10000, 5000

    W_wb, b_wb = params["layer0"]["paper,written_by,author"]
    W_ht, b_ht = params["layer0"]["paper,has_topic,field"]
    W_w, b_w = params["layer1"]["author,writes,paper"]
    W_t, b_t = params["layer1"]["field,topic_of,paper"]

    s_wb, d_wb, e_wb = _pad_edges(edges["paper,written_by,author"], n_author)
    s_ht, d_ht, e_ht = _pad_edges(edges["paper,has_topic,field"], n_field)
    s_w, d_w, e_w = _pad_edges(edges["author,writes,paper"], n_paper)
    s_t, d_t, e_t = _pad_edges(edges["field,topic_of,paper"], n_paper)

    acc_a = _ceil_to(n_author + 8, NS * K)
    acc_f = _ceil_to(n_field + 8, NS * K)
    acc_p = _ceil_to(n_paper + 8, NS * K)
    zf = jnp.zeros((K, D), jnp.float32)

    sum_a, sum_f, deg_a, deg_f = _make_feat2(e_wb, acc_a, e_ht, acc_f, 13, 20)(
        paper, s_wb, d_wb, paper, s_ht, d_ht, zf)
    h_a = _mean_linear(sum_a[:, :n_author],
                       deg_a.reshape(NW, -1).T[:n_author],
                       W_wb, b_wb, n_author, leaky=True)
    h_f = _mean_linear(sum_f[:, :n_field],
                       deg_f.reshape(NW, -1).T[:n_field],
                       W_ht, b_ht, n_field, leaky=True)

    sum_w, sum_t, deg_w, deg_t = _make_feat2(e_w, acc_p, e_t, acc_p, 13, 20)(
        h_a, s_w, d_w, h_f, s_t, d_t, zf)
    return _final_combine(sum_w[:, :n_paper],
                          deg_w.reshape(NW, -1).T[:n_paper], W_w, b_w,
                          sum_t[:, :n_paper],
                          deg_t.reshape(NW, -1).T[:n_paper], W_t, b_t,
                          n_paper)


# speedup vs baseline: 1.1698x; 1.0146x over previous
import functools

import jax
import jax.numpy as jnp
from jax import lax
from jax.experimental import pallas as pl
from jax.experimental.pallas import tpu as pltpu
from jax.experimental.pallas import tpu_sc as plsc

D = 128
K = 128
NC = 2
NS = 16
NW = NC * NS
NB = 4


def _ceil_to(x, m):
    return ((x + m - 1) // m) * m


@functools.lru_cache(maxsize=None)
def _make_feat2(e1, acc1, e2, acc2, c0_frac_num=1, c0_frac_den=2):
    def _split(e):
        tot = e // (NS * K)
        q0 = _ceil_to((tot * c0_frac_num) // c0_frac_den, NB)
        return q0, tot - q0
    qs = (_split(e1), _split(e2))
    zs = (acc1 // NS, acc2 // NS)
    accmax = max(acc1, acc2)
    mesh = plsc.VectorSubcoreMesh(core_axis_name="c", subcore_axis_name="s")
    out_type = [
        jax.ShapeDtypeStruct((NC, acc1, D), jnp.float32),
        jax.ShapeDtypeStruct((NC, acc2, D), jnp.float32),
        jax.ShapeDtypeStruct((NC, NS, acc1), jnp.float32),
        jax.ShapeDtypeStruct((NC, NS, acc2), jnp.float32),
    ]
    scratch = [
        pltpu.VMEM_SHARED((accmax, D), jnp.float32),
        pltpu.VMEM((NB, K), jnp.int32),
        pltpu.VMEM((NB, K), jnp.int32),
        pltpu.VMEM((K, D), jnp.float32),
        pltpu.VMEM((K, D), jnp.float32),
        pltpu.VMEM((accmax,), jnp.float32),
        pltpu.SemaphoreType.DMA,
        pltpu.SemaphoreType.DMA,
        pltpu.SemaphoreType.DMA,
        pltpu.SemaphoreType.DMA,
    ]

    @functools.partial(
        pl.kernel, mesh=mesh, out_type=out_type, scratch_types=scratch,
        compiler_params=pltpu.CompilerParams(needs_layout_passes=False))
    def feat2(t1, s1, d1, t2, s2, d2, zf,
              sum1, sum2, deg1, deg2,
              acc, sblk, dblk, rows0, rows1, hist, sg0, sg1, ss0, ss1):
        c = lax.axis_index("c")
        s = lax.axis_index("s")
        wid = s * NC + c
        ones16 = jnp.ones((16,), jnp.float32)
        zero16 = jnp.zeros((16,), jnp.float32)

        for (table, srcr, dstr, (q0, q1), z, acc_n, sumo, dego) in (
                (t1, s1, d1, qs[0], zs[0], acc1, sum1, deg1),
                (t2, s2, d2, qs[1], zs[1], acc2, sum2, deg2)):
            nzc = z // K
            pltpu.sync_copy(zf, rows0)

            def zbody(i, carry):
                pltpu.sync_copy(rows0, acc.at[pl.ds(s * z + i * K, K)])
                return carry

            lax.fori_loop(0, nzc, zbody, 0)

            def zhist(i, carry):
                hist[pl.ds(i * 16, 16)] = zero16
                return carry

            lax.fori_loop(0, acc_n // 16, zhist, 0)
            plsc.subcore_barrier()
            crow0 = jnp.where(c == 0, s * q0, NS * q0 + s * q1)
            nblk = jnp.where(c == 0, q0 // NB, q1 // NB)

            def body(t, carry):
                crow = crow0 + t * NB
                i0 = pltpu.async_copy(srcr.at[pl.ds(crow, NB)], sblk, sg0)
                i1 = pltpu.async_copy(dstr.at[pl.ds(crow, NB)], dblk, sg1)
                i0.wait()
                i1.wait()
                cprev = None
                for h in range(NB // 2):
                    if cprev is not None:
                        cprev[0].wait()
                    g0 = pltpu.async_copy(table.at[sblk.at[2 * h]],
                                          rows0, sg0)
                    if cprev is not None:
                        cprev[1].wait()
                    g1 = pltpu.async_copy(table.at[sblk.at[2 * h + 1]],
                                          rows1, sg1)
                    for u in (2 * h, 2 * h + 1):
                        for j in range(K // 16):
                            idx16 = dblk[u, pl.ds(j * 16, 16)]
                            plsc.addupdate_scatter(hist, [idx16], ones16)
                    g0.wait()
                    c0 = pltpu.async_copy(rows0, acc.at[dblk.at[2 * h]],
                                          ss0, add=True)
                    g1.wait()
                    c1 = pltpu.async_copy(rows1, acc.at[dblk.at[2 * h + 1]],
                                          ss1, add=True)
                    cprev = (c0, c1)
                cprev[0].wait()
                cprev[1].wait()
                return carry

            lax.fori_loop(0, nblk, body, 0)
            plsc.subcore_barrier()
            pltpu.sync_copy(hist.at[pl.ds(0, acc_n)], dego.at[c, s])

            def obody(i, carry):
                r0 = s * z + i * K
                pltpu.sync_copy(acc.at[pl.ds(r0, K)], rows0)
                pltpu.sync_copy(rows0, sumo.at[c, pl.ds(r0, K)])
                return carry

            lax.fori_loop(0, nzc, obody, 0)
            plsc.subcore_barrier()

    return feat2


def _pad_edges(ei, n_dst):
    src, dst = ei[0], ei[1]
    e = src.shape[0]
    epad = _ceil_to(e, NW * K * NB)
    pad = epad - e
    if pad:
        src = jnp.concatenate([src, jnp.zeros((pad,), jnp.int32)])
        dst = jnp.concatenate([dst, jnp.full((pad,), n_dst, jnp.int32)])
    return src.reshape(epad // K, K), dst.reshape(epad // K, K), epad


def _mean_linear(sums, degs, W, b, n, leaky):
    blk = 1000
    nb = n // blk

    def body(s_ref, d_ref, w_ref, b_ref, o_ref):
        ss = s_ref[...]
        dd = d_ref[...]
        sm = ss[0] + ss[1]
        d = jnp.sum(dd, axis=1)[:, None]
        mean = sm / jnp.maximum(d, 1.0)
        h = jnp.dot(mean, w_ref[...], preferred_element_type=jnp.float32)
        h = h + jnp.where(d > 0, b_ref[...], 0.0)
        if leaky:
            h = jnp.where(h >= 0, h, 0.01 * h)
        o_ref[...] = h

    return pl.pallas_call(
        body,
        grid=(nb,),
        in_specs=[
            pl.BlockSpec((NC, blk, D), lambda i: (0, i, 0)),
            pl.BlockSpec((blk, NW), lambda i: (i, 0)),
            pl.BlockSpec((D, D), lambda i: (0, 0)),
            pl.BlockSpec((1, D), lambda i: (0, 0)),
        ],
        out_specs=pl.BlockSpec((blk, D), lambda i: (i, 0)),
        out_shape=jax.ShapeDtypeStruct((n, D), jnp.float32),
    )(sums, degs, W, b.reshape(1, D))


def _final_combine(sw, dw, Ww, bw, st, dt, Wt, bt, n):
    blk = 1000
    nb = n // blk

    def body(sw_ref, dw_ref, ww_ref, bw_ref, st_ref, dt_ref, wt_ref, bt_ref,
             o_ref):
        out = None
        for s_ref, d_ref, w_ref, b_ref in (
                (sw_ref, dw_ref, ww_ref, bw_ref),
                (st_ref, dt_ref, wt_ref, bt_ref)):
            ss = s_ref[...]
            dd = d_ref[...]
            sm = ss[0] + ss[1]
            d = jnp.sum(dd, axis=1)[:, None]
            mean = sm / jnp.maximum(d, 1.0)
            h = jnp.dot(mean, w_ref[...], preferred_element_type=jnp.float32)
            h = h + jnp.where(d > 0, b_ref[...], 0.0)
            out = h if out is None else out + h
        o_ref[...] = out

    mat = pl.BlockSpec((NC, blk, D), lambda i: (0, i, 0))
    deg = pl.BlockSpec((blk, NW), lambda i: (i, 0))
    wsp = pl.BlockSpec((D, D), lambda i: (0, 0))
    bsp = pl.BlockSpec((1, D), lambda i: (0, 0))
    return pl.pallas_call(
        body,
        grid=(nb,),
        in_specs=[mat, deg, wsp, bsp, mat, deg, wsp, bsp],
        out_specs=pl.BlockSpec((blk, D), lambda i: (i, 0)),
        out_shape=jax.ShapeDtypeStruct((n, D), jnp.float32),
    )(sw, dw, Ww, bw.reshape(1, D), st, dt, Wt, bt.reshape(1, D))


def kernel(embeds, params, edges):
    paper = embeds["paper"]
    n_author, n_paper, n_field = 10000, 10000, 5000

    W_wb, b_wb = params["layer0"]["paper,written_by,author"]
    W_ht, b_ht = params["layer0"]["paper,has_topic,field"]
    W_w, b_w = params["layer1"]["author,writes,paper"]
    W_t, b_t = params["layer1"]["field,topic_of,paper"]

    s_wb, d_wb, e_wb = _pad_edges(edges["paper,written_by,author"], n_author)
    s_ht, d_ht, e_ht = _pad_edges(edges["paper,has_topic,field"], n_field)
    s_w, d_w, e_w = _pad_edges(edges["author,writes,paper"], n_paper)
    s_t, d_t, e_t = _pad_edges(edges["field,topic_of,paper"], n_paper)

    acc_a = _ceil_to(n_author + 8, NS * K)
    acc_f = _ceil_to(n_field + 8, NS * K)
    acc_p = _ceil_to(n_paper + 8, NS * K)
    zf = jnp.zeros((K, D), jnp.float32)

    sum_a, sum_f, deg_a, deg_f = _make_feat2(e_wb, acc_a, e_ht, acc_f, 7, 10)(
        paper, s_wb, d_wb, paper, s_ht, d_ht, zf)
    h_a = _mean_linear(sum_a[:, :n_author],
                       deg_a.reshape(NW, -1).T[:n_author],
                       W_wb, b_wb, n_author, leaky=True)
    h_f = _mean_linear(sum_f[:, :n_field],
                       deg_f.reshape(NW, -1).T[:n_field],
                       W_ht, b_ht, n_field, leaky=True)

    sum_w, sum_t, deg_w, deg_t = _make_feat2(e_w, acc_p, e_t, acc_p, 7, 10)(
        h_a, s_w, d_w, h_f, s_t, d_t, zf)
    return _final_combine(sum_w[:, :n_paper],
                          deg_w.reshape(NW, -1).T[:n_paper], W_w, b_w,
                          sum_t[:, :n_paper],
                          deg_t.reshape(NW, -1).T[:n_paper], W_t, b_t,
                          n_paper)
